# baseline (speedup 1.0000x reference)
import jax
import jax.numpy as jnp
from jax.experimental import pallas as pl
from jax.experimental.pallas import tpu as pltpu

_TEMP = 0.5
_ALPHA = 0.01
_TOP_K = 8
_BT = 1024


def _gate_block(h_ref, g_ref, wt_ref, wg_ref, idx_ref, w_ref, aux_ref,
                ss_ref, cnt_ref, *, nb, bsz, seq_len, n_exp):
    b = pl.program_id(0)
    s = pl.program_id(1)

    @pl.when(jnp.logical_and(b == 0, s == 0))
    def _init():
        ss_ref[...] = jnp.zeros_like(ss_ref)
        cnt_ref[...] = jnp.zeros_like(cnt_ref)

    lh = jnp.dot(h_ref[...], wt_ref[...],
                 preferred_element_type=jnp.float32,
                 precision=jax.lax.Precision.DEFAULT)
    lg = jnp.dot(g_ref[pl.ds(b, 1), :], wg_ref[...],
                 preferred_element_type=jnp.float32,
                 precision=jax.lax.Precision.DEFAULT)
    logits = ((lh + lg / _TEMP) / (1.0 + 1.0 / _TEMP)).T

    m = jnp.max(logits, axis=0, keepdims=True)
    e = jnp.exp(logits - m)
    scores = e / jnp.sum(e, axis=0, keepdims=True)

    row = jax.lax.broadcasted_iota(jnp.int32, scores.shape, 0)
    work = scores
    idx_rows = []
    w_rows = []
    for _ in range(_TOP_K):
        mk = jnp.max(work, axis=0, keepdims=True)
        idxk = jnp.min(jnp.where(work == mk, row, n_exp),
                       axis=0, keepdims=True)
        work = jnp.where(row == idxk, -1.0, work)
        idx_rows.append(idxk)
        w_rows.append(mk)

    w_blk = jnp.concatenate(w_rows, axis=0)
    denom = jnp.sum(w_blk, axis=0, keepdims=True) + 1e-20
    w_ref[...] = (w_blk / denom).T
    idx_ref[...] = jnp.concatenate(idx_rows, axis=0).T

    sel_mask = (work < 0.0).astype(jnp.float32)
    for bb in range(bsz):
        @pl.when(b == bb)
        def _acc(bb=bb):
            ss_ref[bb] += scores
            cnt_ref[bb] += sel_mask

    @pl.when(jnp.logical_and(b == bsz - 1, s == nb - 1))
    def _finish():
        total = jnp.zeros((1, 1), jnp.float32)
        for bb in range(bsz):
            ce = jnp.sum(cnt_ref[bb], axis=1,
                         keepdims=True) / (seq_len * _TOP_K / n_exp)
            ms = jnp.sum(ss_ref[bb], axis=1, keepdims=True) / seq_len
            total = total + jnp.sum(ce * ms, axis=0, keepdims=True)
        aux_ref[...] = total * (_ALPHA / bsz)


def kernel(hidden_states, gene_vectors, weight_token, weight_gene):
    bsz, seq_len, h = hidden_states.shape
    n_exp = weight_token.shape[0]
    gene_len = gene_vectors.shape[-1]
    nb = seq_len // _BT
    n_tok = bsz * seq_len

    hs2d = hidden_states.reshape(n_tok, h)

    grid = (bsz, nb)
    kern = pl.pallas_call(
        lambda *refs: _gate_block(*refs, nb=nb, bsz=bsz, seq_len=seq_len,
                                  n_exp=n_exp),
        grid=grid,
        in_specs=[
            pl.BlockSpec((_BT, h), lambda b, s: (b * nb + s, 0)),
            pl.BlockSpec((bsz, gene_len), lambda b, s: (0, 0)),
            pl.BlockSpec((h, n_exp), lambda b, s: (0, 0)),
            pl.BlockSpec((gene_len, n_exp), lambda b, s: (0, 0)),
        ],
        out_specs=[
            pl.BlockSpec((_BT, _TOP_K), lambda b, s: (b * nb + s, 0)),
            pl.BlockSpec((_BT, _TOP_K), lambda b, s: (b * nb + s, 0)),
            pl.BlockSpec((1, 1), lambda b, s: (0, 0)),
        ],
        out_shape=[
            jax.ShapeDtypeStruct((n_tok, _TOP_K), jnp.int32),
            jax.ShapeDtypeStruct((n_tok, _TOP_K), jnp.float32),
            jax.ShapeDtypeStruct((1, 1), jnp.float32),
        ],
        scratch_shapes=[
            pltpu.VMEM((bsz, n_exp, _BT), jnp.float32),
            pltpu.VMEM((bsz, n_exp, _BT), jnp.float32),
        ],
    )
    topk_idx, topk_weight, aux = kern(hs2d, gene_vectors,
                                      weight_token.T, weight_gene.T)
    return topk_idx, topk_weight, aux[0, 0]


# speedup vs baseline: 1.4543x; 1.4543x over previous
import jax
import jax.numpy as jnp
from jax.experimental import pallas as pl
from jax.experimental.pallas import tpu as pltpu

_TEMP = 0.5
_ALPHA = 0.01
_TOP_K = 8
_BT = 1024


def _gate_block(h_ref, g_ref, wt_ref, wg_ref, idx_ref, w_ref, aux_ref,
                ss_ref, cnt_ref, *, nb, bsz, seq_len, n_exp):
    b = pl.program_id(0)
    s = pl.program_id(1)

    @pl.when(jnp.logical_and(b == 0, s == 0))
    def _init():
        ss_ref[...] = jnp.zeros_like(ss_ref)
        cnt_ref[...] = jnp.zeros_like(cnt_ref)

    dn = (((1,), (1,)), ((), ()))
    lh = jax.lax.dot_general(h_ref[...], wt_ref[...], dn,
                             preferred_element_type=jnp.float32,
                             precision=jax.lax.Precision.DEFAULT)
    lg = jax.lax.dot_general(g_ref[pl.ds(b, 1), :], wg_ref[...], dn,
                             preferred_element_type=jnp.float32,
                             precision=jax.lax.Precision.DEFAULT)
    logits = ((lh + lg / _TEMP) / (1.0 + 1.0 / _TEMP)).T

    m = jnp.max(logits, axis=0, keepdims=True)
    e = jnp.exp(logits - m)
    scores = e / jnp.sum(e, axis=0, keepdims=True)

    row = jax.lax.broadcasted_iota(jnp.int32, scores.shape, 0)
    work = scores
    idx_rows = []
    w_rows = []
    for _ in range(_TOP_K):
        mk = jnp.max(work, axis=0, keepdims=True)
        idxk = jnp.min(jnp.where(work == mk, row, n_exp),
                       axis=0, keepdims=True)
        work = jnp.where(row == idxk, -1.0, work)
        idx_rows.append(idxk)
        w_rows.append(mk)

    w_blk = jnp.concatenate(w_rows, axis=0)
    denom = jnp.sum(w_blk, axis=0, keepdims=True) + 1e-20
    w_ref[...] = w_blk / denom
    idx_ref[...] = jnp.concatenate(idx_rows, axis=0)

    sel_mask = (work < 0.0).astype(jnp.float32)
    for bb in range(bsz):
        @pl.when(b == bb)
        def _acc(bb=bb):
            ss_ref[bb] += scores
            cnt_ref[bb] += sel_mask

    @pl.when(jnp.logical_and(b == bsz - 1, s == nb - 1))
    def _finish():
        total = jnp.zeros((1, 1), jnp.float32)
        for bb in range(bsz):
            ce = jnp.sum(cnt_ref[bb], axis=1,
                         keepdims=True) / (seq_len * _TOP_K / n_exp)
            ms = jnp.sum(ss_ref[bb], axis=1, keepdims=True) / seq_len
            total = total + jnp.sum(ce * ms, axis=0, keepdims=True)
        aux_ref[...] = total * (_ALPHA / bsz)


def kernel(hidden_states, gene_vectors, weight_token, weight_gene):
    bsz, seq_len, h = hidden_states.shape
    n_exp = weight_token.shape[0]
    gene_len = gene_vectors.shape[-1]
    nb = seq_len // _BT
    n_tok = bsz * seq_len

    hs2d = hidden_states.reshape(n_tok, h)

    grid = (bsz, nb)
    kern = pl.pallas_call(
        lambda *refs: _gate_block(*refs, nb=nb, bsz=bsz, seq_len=seq_len,
                                  n_exp=n_exp),
        grid=grid,
        in_specs=[
            pl.BlockSpec((_BT, h), lambda b, s: (b * nb + s, 0)),
            pl.BlockSpec((bsz, gene_len), lambda b, s: (0, 0)),
            pl.BlockSpec((n_exp, h), lambda b, s: (0, 0)),
            pl.BlockSpec((n_exp, gene_len), lambda b, s: (0, 0)),
        ],
        out_specs=[
            pl.BlockSpec((_TOP_K, _BT), lambda b, s: (0, b * nb + s)),
            pl.BlockSpec((_TOP_K, _BT), lambda b, s: (0, b * nb + s)),
            pl.BlockSpec((1, 1), lambda b, s: (0, 0)),
        ],
        out_shape=[
            jax.ShapeDtypeStruct((_TOP_K, n_tok), jnp.int32),
            jax.ShapeDtypeStruct((_TOP_K, n_tok), jnp.float32),
            jax.ShapeDtypeStruct((1, 1), jnp.float32),
        ],
        scratch_shapes=[
            pltpu.VMEM((bsz, n_exp, _BT), jnp.float32),
            pltpu.VMEM((bsz, n_exp, _BT), jnp.float32),
        ],
    )
    idx_t, w_t, aux = kern(hs2d, gene_vectors, weight_token, weight_gene)
    return idx_t.T, w_t.T, aux[0, 0]
